# initial kernel scaffold (unmeasured)
import functools

import jax
import jax.numpy as jnp
from jax import lax
from jax.experimental import pallas as pl
from jax.experimental.pallas import tpu as pltpu


def kernel(x, dy):
    d = x.shape[1]
    f = dy.shape[1]
    half = d // 2

    def body(x_ref, dy_ref, out_ref, send_buf, recv_buf, send_sem, recv_sem):
        my_x = lax.axis_index("x")
        my_y = lax.axis_index("y")
        my_z = lax.axis_index("z")
        partner = (1 - my_x, my_y, my_z)

        xb = x_ref[...].astype(jnp.bfloat16)
        dyb = dy_ref[...].astype(jnp.bfloat16)
        p = lax.dot_general(
            xb, dyb, (((0,), (0,)), ((), ())),
            preferred_element_type=jnp.float32,
        )

        keep = lax.dynamic_slice(p, (my_x * half, 0), (half, f))
        other = lax.dynamic_slice(p, ((1 - my_x) * half, 0), (half, f))
        send_buf[...] = other.astype(jnp.bfloat16)

        rdma = pltpu.make_async_remote_copy(
            src_ref=send_buf,
            dst_ref=recv_buf,
            send_sem=send_sem,
            recv_sem=recv_sem,
            device_id=partner,
            device_id_type=pl.DeviceIdType.MESH,
        )
        rdma.start()
        rdma.wait()

        out_ref[...] = keep + recv_buf[...].astype(jnp.float32)

        @functools.partial(pl.run_scoped, sem=pltpu.SemaphoreType.REGULAR)
        def _(sem):
            pl.semaphore_signal(
                sem, inc=1, device_id=partner,
                device_id_type=pl.DeviceIdType.MESH,
            )
            pl.semaphore_wait(sem, 1)

    return pl.pallas_call(
        body,
        out_shape=jax.ShapeDtypeStruct((half, f), jnp.float32),
        in_specs=[
            pl.BlockSpec(memory_space=pltpu.VMEM),
            pl.BlockSpec(memory_space=pltpu.VMEM),
        ],
        out_specs=pl.BlockSpec(memory_space=pltpu.VMEM),
        scratch_shapes=[
            pltpu.VMEM((half, f), jnp.bfloat16),
            pltpu.VMEM((half, f), jnp.bfloat16),
            pltpu.SemaphoreType.DMA,
            pltpu.SemaphoreType.DMA,
        ],
    )(x, dy)


# baseline (device time: 80053 ns/iter reference)
import functools

import jax
import jax.numpy as jnp
from jax import lax
from jax.experimental import pallas as pl
from jax.experimental.pallas import tpu as pltpu


def kernel(x, dy):
    d = x.shape[1]
    f = dy.shape[1]
    half = d // 2

    def body(x_ref, dy_ref, out_ref, send_buf, recv_buf,
             send_sem, recv_sem):
        my_x = lax.axis_index("x")
        my_y = lax.axis_index("y")
        my_z = lax.axis_index("z")
        partner = (1 - my_x, my_y, my_z)

        dyb = dy_ref[...].astype(jnp.bfloat16)

        def half_gemm(row_start):
            xb = x_ref[:, pl.ds(row_start, half)].astype(jnp.bfloat16)
            return lax.dot_general(
                xb, dyb, (((0,), (0,)), ((), ())),
                preferred_element_type=jnp.float32,
            )

        send_buf[...] = half_gemm((1 - my_x) * half).astype(jnp.bfloat16)

        rdma = pltpu.make_async_remote_copy(
            src_ref=send_buf,
            dst_ref=recv_buf,
            send_sem=send_sem,
            recv_sem=recv_sem,
            device_id=partner,
            device_id_type=pl.DeviceIdType.MESH,
        )
        rdma.start()

        keep = half_gemm(my_x * half)

        rdma.wait()
        out_ref[...] = keep + recv_buf[...].astype(jnp.float32)

        @functools.partial(pl.run_scoped, sem=pltpu.SemaphoreType.REGULAR)
        def _(sem):
            pl.semaphore_signal(
                sem, inc=1, device_id=partner,
                device_id_type=pl.DeviceIdType.MESH,
            )
            pl.semaphore_wait(sem, 1)

    return pl.pallas_call(
        body,
        out_shape=jax.ShapeDtypeStruct((half, f), jnp.float32),
        in_specs=[
            pl.BlockSpec(memory_space=pltpu.VMEM),
            pl.BlockSpec(memory_space=pltpu.VMEM),
        ],
        out_specs=pl.BlockSpec(memory_space=pltpu.VMEM),
        scratch_shapes=[
            pltpu.VMEM((half, f), jnp.bfloat16),
            pltpu.VMEM((half, f), jnp.bfloat16),
            pltpu.SemaphoreType.DMA,
            pltpu.SemaphoreType.DMA,
        ],
        compiler_params=pltpu.CompilerParams(
            vmem_limit_bytes=100 * 1024 * 1024,
        ),
    )(x, dy)


# device time: 63599 ns/iter; 1.2587x vs baseline; 1.2587x over previous
import functools

import jax
import jax.numpy as jnp
from jax import lax
from jax.experimental import pallas as pl
from jax.experimental.pallas import tpu as pltpu

NC = 4


def kernel(x, dy):
    d = x.shape[1]
    f = dy.shape[1]
    half = d // 2
    blk = d // 8
    fc = f // NC

    def body(x_ref, dy_ref, out_ref, g_ref, send_buf, recv_buf,
             send_sems, recv_sems):
        my_x = lax.axis_index("x")
        my_y = lax.axis_index("y")
        my_z = lax.axis_index("z")
        q = 2 * my_y + my_z
        x_partner = (1 - my_x, my_y, my_z)
        z_partner = (my_x, my_y, 1 - my_z)
        y_partner = (my_x, 1 - my_y, my_z)

        dyb = dy_ref[...].astype(jnp.bfloat16)

        def blk_gemm(row_start, c):
            xb = x_ref[:, pl.ds(row_start, blk)].astype(jnp.bfloat16)
            return lax.dot_general(
                xb, dyb[:, c * fc:(c + 1) * fc], (((0,), (0,)), ((), ())),
                preferred_element_type=jnp.float32,
            )

        keep_start = my_x * half + blk * q
        send_start = (1 - my_x) * half + blk * q

        def rdma(phase, c, src, dst, dev):
            return pltpu.make_async_remote_copy(
                src_ref=src, dst_ref=dst,
                send_sem=send_sems.at[phase, c],
                recv_sem=recv_sems.at[phase, c],
                device_id=dev, device_id_type=pl.DeviceIdType.MESH,
            )

        r1 = [rdma(0, c, send_buf.at[c], recv_buf.at[c], x_partner)
              for c in range(NC)]
        r2 = [rdma(1, c,
                   g_ref.at[c, pl.ds(blk * q, blk), :],
                   g_ref.at[c, pl.ds(blk * q, blk), :],
                   z_partner)
              for c in range(NC)]
        r3 = [rdma(2, c,
                   g_ref.at[c, pl.ds(2 * blk * my_y, 2 * blk), :],
                   g_ref.at[c, pl.ds(2 * blk * my_y, 2 * blk), :],
                   y_partner)
              for c in range(NC)]

        for c in range(NC):
            send_buf[c] = blk_gemm(send_start, c).astype(jnp.bfloat16)
            r1[c].start()
            g_ref[c, pl.ds(blk * q, blk), :] = (
                blk_gemm(keep_start, c).astype(jnp.bfloat16))

        for c in range(NC):
            r1[c].wait()
            g_ref[c, pl.ds(blk * q, blk), :] = (
                g_ref[c, pl.ds(blk * q, blk), :].astype(jnp.float32)
                + recv_buf[c].astype(jnp.float32)
            ).astype(jnp.bfloat16)
            r2[c].start()

        for c in range(NC):
            r2[c].wait()
            r3[c].start()

        for c in range(NC):
            r3[c].wait()
            out_ref[:, c * fc:(c + 1) * fc] = g_ref[c].astype(jnp.float32)

        @functools.partial(pl.run_scoped, sem=pltpu.SemaphoreType.REGULAR)
        def _(sem):
            for dev in (x_partner, z_partner, y_partner):
                pl.semaphore_signal(
                    sem, inc=1, device_id=dev,
                    device_id_type=pl.DeviceIdType.MESH,
                )
            pl.semaphore_wait(sem, 3)

    return pl.pallas_call(
        body,
        out_shape=jax.ShapeDtypeStruct((half, f), jnp.float32),
        in_specs=[
            pl.BlockSpec(memory_space=pltpu.VMEM),
            pl.BlockSpec(memory_space=pltpu.VMEM),
        ],
        out_specs=pl.BlockSpec(memory_space=pltpu.VMEM),
        scratch_shapes=[
            pltpu.VMEM((NC, half, fc), jnp.bfloat16),
            pltpu.VMEM((NC, blk, fc), jnp.bfloat16),
            pltpu.VMEM((NC, blk, fc), jnp.bfloat16),
            pltpu.SemaphoreType.DMA((3, NC)),
            pltpu.SemaphoreType.DMA((3, NC)),
        ],
        compiler_params=pltpu.CompilerParams(
            vmem_limit_bytes=100 * 1024 * 1024,
        ),
    )(x, dy)


# device time: 45093 ns/iter; 1.7753x vs baseline; 1.4104x over previous
import functools

import jax
import jax.numpy as jnp
from jax import lax
from jax.experimental import pallas as pl
from jax.experimental.pallas import tpu as pltpu

NC = 8


def kernel(x, dy):
    d = x.shape[1]
    f = dy.shape[1]
    half = d // 2
    blk = d // 8
    fc = f // NC
    fh = fc // 2

    def body(x_ref, dy_ref, out_ref, g_ref, send_buf, recv_buf,
             send_sems, recv_sems):
        my_x = lax.axis_index("x")
        my_y = lax.axis_index("y")
        my_z = lax.axis_index("z")
        q = 2 * my_y + my_z
        q_from_z = 2 * my_y + (1 - my_z)
        q_from_y = 2 * (1 - my_y) + my_z
        x_partner = (1 - my_x, my_y, my_z)
        z_partner = (my_x, my_y, 1 - my_z)
        y_partner = (my_x, 1 - my_y, my_z)

        dyb = dy_ref[...].astype(jnp.bfloat16)

        def blk_gemm(row_start, c):
            xb = x_ref[:, pl.ds(row_start, blk)].astype(jnp.bfloat16)
            return lax.dot_general(
                xb, dyb[:, c * fc:(c + 1) * fc], (((0,), (0,)), ((), ())),
                preferred_element_type=jnp.float32,
            )

        keep_start = my_x * half + blk * q
        send_start = (1 - my_x) * half + blk * q

        def rdma(phase, c, src, dst, dev):
            return pltpu.make_async_remote_copy(
                src_ref=src, dst_ref=dst,
                send_sem=send_sems.at[phase, c],
                recv_sem=recv_sems.at[phase, c],
                device_id=dev, device_id_type=pl.DeviceIdType.MESH,
            )

        rx = [rdma(0, c, send_buf.at[c], recv_buf.at[c], x_partner)
              for c in range(NC)]
        rz1 = [rdma(1, c,
                    g_ref.at[c, pl.ds(blk * q, blk), :],
                    g_ref.at[c, pl.ds(blk * q, blk), :],
                    z_partner)
               for c in range(NC)]
        ry1 = [rdma(2, c,
                    g_ref.at[c, pl.ds(blk * q, blk), :],
                    g_ref.at[c, pl.ds(blk * q, blk), :],
                    y_partner)
               for c in range(NC)]
        bh = blk // 2
        rz2 = [rdma(3, c,
                    g_ref.at[c, pl.ds(blk * q_from_y, bh), :],
                    g_ref.at[c, pl.ds(blk * q_from_y, bh), :],
                    z_partner)
               for c in range(NC)]
        ry2 = [rdma(4, c,
                    g_ref.at[c, pl.ds(blk * q_from_z + bh, bh), :],
                    g_ref.at[c, pl.ds(blk * q_from_z + bh, bh), :],
                    y_partner)
               for c in range(NC)]

        barrier_sem = pltpu.get_barrier_semaphore()
        for dev in (x_partner, z_partner, y_partner):
            pl.semaphore_signal(
                barrier_sem, inc=1, device_id=dev,
                device_id_type=pl.DeviceIdType.MESH,
            )
        pl.semaphore_wait(barrier_sem, 3)

        for c in range(NC):
            send_buf[c] = blk_gemm(send_start, c).astype(jnp.bfloat16)
            rx[c].start()

        for c in range(NC):
            g_ref[c, pl.ds(blk * q, blk), :] = (
                blk_gemm(keep_start, c).astype(jnp.bfloat16))
            rx[c].wait()
            g_ref[c, pl.ds(blk * q, blk), :] = (
                g_ref[c, pl.ds(blk * q, blk), :].astype(jnp.float32)
                + recv_buf[c].astype(jnp.float32)
            ).astype(jnp.bfloat16)
            rz1[c].start()
            ry1[c].start()

        for c in range(NC):
            rz1[c].wait()
            ry1[c].wait()
            rz2[c].start()
            ry2[c].start()

        for c in range(NC):
            rz2[c].wait()
            ry2[c].wait()
            out_ref[:, c * fc:(c + 1) * fc] = g_ref[c].astype(jnp.float32)

        @functools.partial(pl.run_scoped, sem=pltpu.SemaphoreType.REGULAR)
        def _(sem):
            for dev in (x_partner, z_partner, y_partner):
                pl.semaphore_signal(
                    sem, inc=1, device_id=dev,
                    device_id_type=pl.DeviceIdType.MESH,
                )
            pl.semaphore_wait(sem, 3)

    return pl.pallas_call(
        body,
        out_shape=jax.ShapeDtypeStruct((half, f), jnp.float32),
        in_specs=[
            pl.BlockSpec(memory_space=pltpu.VMEM),
            pl.BlockSpec(memory_space=pltpu.VMEM),
        ],
        out_specs=pl.BlockSpec(memory_space=pltpu.VMEM),
        scratch_shapes=[
            pltpu.VMEM((NC, half, fc), jnp.bfloat16),
            pltpu.VMEM((NC, blk, fc), jnp.bfloat16),
            pltpu.VMEM((NC, blk, fc), jnp.bfloat16),
            pltpu.SemaphoreType.DMA((5, NC)),
            pltpu.SemaphoreType.DMA((5, NC)),
        ],
        compiler_params=pltpu.CompilerParams(
            vmem_limit_bytes=100 * 1024 * 1024,
            collective_id=0,
        ),
    )(x, dy)
